# baseline (speedup 1.0000x reference)
import functools

import jax
import jax.numpy as jnp
from jax import lax
from jax.experimental import pallas as pl
from jax.experimental.pallas import tpu as pltpu
from jax.experimental.pallas import tpu_sc as plsc

_BETA = 0.2
_BM = 128
_NB = 96
_ROWS_PAD = _NB * _BM

_SC = plsc.get_sparse_core_info()
_NW = _SC.num_cores * _SC.num_subcores


def _router_body(hs_ref, gw_ref, logits_ref, meta_ref, counts_ref, carry_ref):
    i = pl.program_id(0)

    @pl.when(i == 0)
    def _():
        carry_ref[...] = jnp.zeros_like(carry_ref)

    x = hs_ref[...]
    logits = jax.lax.dot_general(
        x, gw_ref[...], (((1,), (1,)), ((), ())),
        preferred_element_type=jnp.float32)
    logits_ref[...] = logits

    mx = jnp.max(logits, axis=1, keepdims=True)
    ex = jnp.exp(logits - mx)
    p = ex / jnp.sum(ex, axis=1, keepdims=True)

    bm, e = p.shape
    idx = jax.lax.broadcasted_iota(jnp.int32, (bm, e), 1)
    p1 = jnp.max(p, axis=1, keepdims=True)
    e0 = jnp.min(jnp.where(p == p1, idx, e), axis=1, keepdims=True)
    pm = jnp.where(idx == e0, -jnp.inf, p)
    p2 = jnp.max(pm, axis=1, keepdims=True)
    e1 = jnp.min(jnp.where(pm == p2, idx, e), axis=1, keepdims=True)

    skip = p2 < _BETA * p1
    denom = p1 + jnp.where(skip, 0.0, p2)
    w0 = p1 / denom
    w1 = jnp.where(skip, 0.0, p2 / denom)

    oh0 = (idx == e0).astype(jnp.float32)
    oh1 = (idx == e1).astype(jnp.float32)
    ohall = oh0 + oh1
    ri = jax.lax.broadcasted_iota(jnp.int32, (bm, bm), 0)
    ci = jax.lax.broadcasted_iota(jnp.int32, (bm, bm), 1)
    tri = (ri > ci).astype(jnp.float32)
    csum_excl = jax.lax.dot_general(tri, ohall, (((1,), (0,)), ((), ())),
                                    preferred_element_type=jnp.float32)
    tot = carry_ref[0:1, :] + csum_excl
    rank0 = jnp.sum(tot * oh0, axis=1, keepdims=True)
    rank1 = jnp.sum(tot * oh1, axis=1, keepdims=True)
    newc = carry_ref[0:1, :] + jnp.sum(ohall, axis=0, keepdims=True)
    carry_ref[0:1, :] = newc
    counts_ref[0:1, :] = newc

    col = jax.lax.broadcasted_iota(jnp.int32, (bm, meta_ref.shape[1]), 1)
    meta = (w0 * (col == 0) + w1 * (col == 1)
            + e0.astype(jnp.float32) * (col == 2)
            + e1.astype(jnp.float32) * (col == 3)
            + rank0 * (col == 4) + rank1 * (col == 5))
    meta_ref[...] = meta


def _ffn_body(g_ref, bs_ref, v_ref, x_ref, w1_ref, w3_ref, w2_ref, out_ref):
    @pl.when(v_ref[pl.program_id(0)] == 1)
    def _():
        x = x_ref[...]
        a = jax.lax.dot_general(x, w1_ref[0], (((1,), (1,)), ((), ())),
                                preferred_element_type=jnp.float32)
        b = jax.lax.dot_general(x, w3_ref[0], (((1,), (1,)), ((), ())),
                                preferred_element_type=jnp.float32)
        h = (a * jax.nn.sigmoid(a)) * b
        out_ref[...] = jax.lax.dot_general(
            h, w2_ref[0], (((1,), (1,)), ((), ())),
            preferred_element_type=jnp.float32)


def _dispatch_body(hs_ref, pp32_ref, x_pad_ref, idx0_v, idx1_v, rows_v, sem):
    wid = lax.axis_index("s") * _SC.num_cores + lax.axis_index("c")
    tw = rows_v.shape[0]
    tc = idx0_v.shape[0]
    base = wid * tw
    pltpu.sync_copy(hs_ref.at[pl.ds(base, tw)], rows_v)
    for h in range(2):
        pltpu.sync_copy(pp32_ref.at[4 * wid + 2 * h], idx0_v)
        pltpu.sync_copy(pp32_ref.at[4 * wid + 2 * h + 1], idx1_v)
        src = rows_v.at[pl.ds(h * tc, tc)]
        pltpu.async_copy(src, x_pad_ref.at[idx0_v], sem).wait()
        pltpu.async_copy(src, x_pad_ref.at[idx1_v], sem).wait()


def _combine_sc_body(out_pad_ref, pp32_ref, wb0_ref, wb1_ref, final_ref,
                     idx0_v, idx1_v, bufa_v, bufb_v, wv0_v, wv1_v, sem):
    wid = lax.axis_index("s") * _SC.num_cores + lax.axis_index("c")
    tc, d = bufa_v.shape
    nch = d // 16
    for h in range(2):
        base = wid * 2 * tc + h * tc
        pltpu.sync_copy(pp32_ref.at[4 * wid + 2 * h], idx0_v)
        pltpu.sync_copy(pp32_ref.at[4 * wid + 2 * h + 1], idx1_v)
        pltpu.async_copy(out_pad_ref.at[idx0_v], bufa_v, sem).wait()
        pltpu.async_copy(out_pad_ref.at[idx1_v], bufb_v, sem).wait()
        pltpu.sync_copy(wb0_ref.at[pl.ds(base, tc)], wv0_v)
        pltpu.sync_copy(wb1_ref.at[pl.ds(base, tc)], wv1_v)

        def _row(j, _):
            wa = wv0_v[j, :]
            wb = wv1_v[j, :]

            def _chunk(c, _):
                a = bufa_v[j, pl.ds(c * 16, 16)]
                b = bufb_v[j, pl.ds(c * 16, 16)]
                bufa_v[j, pl.ds(c * 16, 16)] = a * wa + b * wb
                return 0

            return lax.fori_loop(0, nch, _chunk, 0)

        lax.fori_loop(0, tc, _row, 0)
        pltpu.sync_copy(bufa_v, final_ref.at[pl.ds(base, tc)])


def kernel(hidden_states, gate_w, w1, w3, w2):
    batch, seq, d = hidden_states.shape
    n_tok = batch * seq
    e_num = gate_w.shape[0]
    f = w1.shape[1]
    hs = hidden_states.reshape(n_tok, d)

    bm_r = 256
    logits, meta, counts_f = pl.pallas_call(
        _router_body,
        grid=(n_tok // bm_r,),
        in_specs=[
            pl.BlockSpec((bm_r, d), lambda i: (i, 0)),
            pl.BlockSpec((e_num, d), lambda i: (0, 0)),
        ],
        out_specs=[
            pl.BlockSpec((bm_r, e_num), lambda i: (i, 0)),
            pl.BlockSpec((bm_r, 128), lambda i: (i, 0)),
            pl.BlockSpec((8, e_num), lambda i: (0, 0)),
        ],
        out_shape=[
            jax.ShapeDtypeStruct((n_tok, e_num), jnp.float32),
            jax.ShapeDtypeStruct((n_tok, 128), jnp.float32),
            jax.ShapeDtypeStruct((8, e_num), jnp.float32),
        ],
        scratch_shapes=[pltpu.VMEM((8, e_num), jnp.float32)],
    )(hs, gate_w)

    w01 = meta[:, :2]
    e_all = meta[:, 2:4].astype(jnp.int32).reshape(2 * n_tok)

    n_asg = 2 * n_tok
    rank = meta[:, 4:6].astype(jnp.int32).reshape(n_asg)
    counts = counts_f[0].astype(jnp.int32)

    blocks_per = (counts + _BM - 1) // _BM
    total_blocks = jnp.sum(blocks_per)
    pstart = (jnp.cumsum(blocks_per) - blocks_per) * _BM

    g_map = jnp.repeat(jnp.arange(e_num, dtype=jnp.int32), blocks_per,
                       total_repeat_length=_NB)
    bidx = jnp.arange(_NB, dtype=jnp.int32)
    bs_map = jnp.minimum(bidx, total_blocks - 1)
    v_map = (bidx < total_blocks).astype(jnp.int32)

    pp = pstart[e_all] + rank
    tw = n_tok // _NW
    tc = tw // 2
    pp32 = (pp.reshape(_NW, 2, tc, 2).transpose(0, 1, 3, 2)
            .reshape(4 * _NW, tc))

    mesh = plsc.VectorSubcoreMesh(core_axis_name="c", subcore_axis_name="s")
    x_pad = pl.kernel(
        _dispatch_body,
        out_type=jax.ShapeDtypeStruct((_ROWS_PAD, d), jnp.float32),
        mesh=mesh,
        scratch_types=[
            pltpu.VMEM((tc,), jnp.int32),
            pltpu.VMEM((tc,), jnp.int32),
            pltpu.VMEM((tw, d), jnp.float32),
            pltpu.SemaphoreType.DMA,
        ],
    )(hs, pp32)

    out_pad = pl.pallas_call(
        _ffn_body,
        grid_spec=pltpu.PrefetchScalarGridSpec(
            num_scalar_prefetch=3,
            grid=(_NB,),
            in_specs=[
                pl.BlockSpec((_BM, d), lambda i, g, bs, v: (bs[i], 0)),
                pl.BlockSpec((1, f, d), lambda i, g, bs, v: (g[i], 0, 0)),
                pl.BlockSpec((1, f, d), lambda i, g, bs, v: (g[i], 0, 0)),
                pl.BlockSpec((1, d, f), lambda i, g, bs, v: (g[i], 0, 0)),
            ],
            out_specs=pl.BlockSpec((_BM, d), lambda i, g, bs, v: (bs[i], 0)),
        ),
        out_shape=jax.ShapeDtypeStruct((_ROWS_PAD, d), jnp.float32),
    )(g_map, bs_map, v_map, x_pad, w1, w3, w2)

    wb0 = jnp.broadcast_to(w01[:, 0:1], (n_tok, 16))
    wb1 = jnp.broadcast_to(w01[:, 1:2], (n_tok, 16))
    final = pl.kernel(
        _combine_sc_body,
        out_type=jax.ShapeDtypeStruct((n_tok, d), jnp.float32),
        mesh=plsc.VectorSubcoreMesh(core_axis_name="c",
                                    subcore_axis_name="s"),
        scratch_types=[
            pltpu.VMEM((tc,), jnp.int32),
            pltpu.VMEM((tc,), jnp.int32),
            pltpu.VMEM((tc, d), jnp.float32),
            pltpu.VMEM((tc, d), jnp.float32),
            pltpu.VMEM((tc, 16), jnp.float32),
            pltpu.VMEM((tc, 16), jnp.float32),
            pltpu.SemaphoreType.DMA,
        ],
    )(out_pad, pp32, wb0, wb1)

    return final.reshape(batch, seq, d), logits


# speedup vs baseline: 1.0117x; 1.0117x over previous
import functools

import jax
import jax.numpy as jnp
from jax import lax
from jax.experimental import pallas as pl
from jax.experimental.pallas import tpu as pltpu
from jax.experimental.pallas import tpu_sc as plsc

_BETA = 0.2
_BM = 128
_NB = 96
_ROWS_PAD = _NB * _BM

_SC = plsc.get_sparse_core_info()
_NW = _SC.num_cores * _SC.num_subcores


def _router_body(hs_ref, gw_ref, logits_ref, meta_ref, counts_ref, carry_ref):
    i = pl.program_id(0)

    @pl.when(i == 0)
    def _():
        carry_ref[...] = jnp.zeros_like(carry_ref)

    x = hs_ref[...]
    logits = jax.lax.dot_general(
        x, gw_ref[...], (((1,), (1,)), ((), ())),
        preferred_element_type=jnp.float32)
    logits_ref[...] = logits

    mx = jnp.max(logits, axis=1, keepdims=True)
    ex = jnp.exp(logits - mx)
    p = ex / jnp.sum(ex, axis=1, keepdims=True)

    bm, e = p.shape
    idx = jax.lax.broadcasted_iota(jnp.int32, (bm, e), 1)
    p1 = jnp.max(p, axis=1, keepdims=True)
    e0 = jnp.min(jnp.where(p == p1, idx, e), axis=1, keepdims=True)
    pm = jnp.where(idx == e0, -jnp.inf, p)
    p2 = jnp.max(pm, axis=1, keepdims=True)
    e1 = jnp.min(jnp.where(pm == p2, idx, e), axis=1, keepdims=True)

    skip = p2 < _BETA * p1
    denom = p1 + jnp.where(skip, 0.0, p2)
    w0 = p1 / denom
    w1 = jnp.where(skip, 0.0, p2 / denom)

    oh0 = (idx == e0).astype(jnp.float32)
    oh1 = (idx == e1).astype(jnp.float32)
    ohall = oh0 + oh1
    ri = jax.lax.broadcasted_iota(jnp.int32, (bm, bm), 0)
    ci = jax.lax.broadcasted_iota(jnp.int32, (bm, bm), 1)
    tri = (ri > ci).astype(jnp.float32)
    csum_excl = jax.lax.dot_general(tri, ohall, (((1,), (0,)), ((), ())),
                                    preferred_element_type=jnp.float32)
    tot = carry_ref[0:1, :] + csum_excl
    rank0 = jnp.sum(tot * oh0, axis=1, keepdims=True)
    rank1 = jnp.sum(tot * oh1, axis=1, keepdims=True)
    newc = carry_ref[0:1, :] + jnp.sum(ohall, axis=0, keepdims=True)
    carry_ref[0:1, :] = newc
    counts_ref[0:1, :] = newc

    col = jax.lax.broadcasted_iota(jnp.int32, (bm, meta_ref.shape[1]), 1)
    meta = (w0 * (col == 0) + w1 * (col == 1)
            + e0.astype(jnp.float32) * (col == 2)
            + e1.astype(jnp.float32) * (col == 3)
            + rank0 * (col == 4) + rank1 * (col == 5))
    meta_ref[...] = meta


def _ffn_body(g_ref, bs_ref, v_ref, x_ref, w1_ref, w3_ref, w2_ref, out_ref):
    @pl.when(v_ref[pl.program_id(0)] == 1)
    def _():
        x = x_ref[...]
        a = jax.lax.dot_general(x, w1_ref[0], (((1,), (1,)), ((), ())),
                                preferred_element_type=jnp.float32)
        b = jax.lax.dot_general(x, w3_ref[0], (((1,), (1,)), ((), ())),
                                preferred_element_type=jnp.float32)
        h = (a * jax.nn.sigmoid(a)) * b
        out_ref[...] = jax.lax.dot_general(
            h, w2_ref[0], (((1,), (1,)), ((), ())),
            preferred_element_type=jnp.float32)


def _dispatch_body(hs_ref, pp32_ref, x_pad_ref, idx_v, rows_v, sem):
    wid = lax.axis_index("s") * _SC.num_cores + lax.axis_index("c")
    tw = rows_v.shape[0]
    tc = idx_v.shape[1]
    base = wid * tw
    pltpu.sync_copy(hs_ref.at[pl.ds(base, tw)], rows_v)
    pltpu.sync_copy(pp32_ref.at[pl.ds(4 * wid, 4)], idx_v)
    copies = []
    for h in range(2):
        src = rows_v.at[pl.ds(h * tc, tc)]
        copies.append(pltpu.async_copy(src, x_pad_ref.at[idx_v.at[2 * h]],
                                       sem))
        copies.append(pltpu.async_copy(src, x_pad_ref.at[idx_v.at[2 * h + 1]],
                                       sem))
    for c in copies:
        c.wait()


def _combine_sc_body(out_pad_ref, pp32_ref, wb0_ref, wb1_ref, final_ref,
                     idx0_v, idx1_v, bufa_v, bufb_v, wv0_v, wv1_v, sem):
    wid = lax.axis_index("s") * _SC.num_cores + lax.axis_index("c")
    tc, d = bufa_v.shape
    nch = d // 16
    for h in range(2):
        base = wid * 2 * tc + h * tc
        pltpu.sync_copy(pp32_ref.at[4 * wid + 2 * h], idx0_v)
        pltpu.sync_copy(pp32_ref.at[4 * wid + 2 * h + 1], idx1_v)
        ca = pltpu.async_copy(out_pad_ref.at[idx0_v], bufa_v, sem)
        cb = pltpu.async_copy(out_pad_ref.at[idx1_v], bufb_v, sem)
        pltpu.sync_copy(wb0_ref.at[pl.ds(base, tc)], wv0_v)
        pltpu.sync_copy(wb1_ref.at[pl.ds(base, tc)], wv1_v)
        ca.wait()
        cb.wait()

        def _row(j, _):
            wa = wv0_v[j, :]
            wb = wv1_v[j, :]

            def _chunk(c, _):
                a = bufa_v[j, pl.ds(c * 16, 16)]
                b = bufb_v[j, pl.ds(c * 16, 16)]
                bufa_v[j, pl.ds(c * 16, 16)] = a * wa + b * wb
                return 0

            return lax.fori_loop(0, nch, _chunk, 0)

        lax.fori_loop(0, tc, _row, 0)
        pltpu.sync_copy(bufa_v, final_ref.at[pl.ds(base, tc)])


def kernel(hidden_states, gate_w, w1, w3, w2):
    batch, seq, d = hidden_states.shape
    n_tok = batch * seq
    e_num = gate_w.shape[0]
    f = w1.shape[1]
    hs = hidden_states.reshape(n_tok, d)

    bm_r = 256
    logits, meta, counts_f = pl.pallas_call(
        _router_body,
        grid=(n_tok // bm_r,),
        in_specs=[
            pl.BlockSpec((bm_r, d), lambda i: (i, 0)),
            pl.BlockSpec((e_num, d), lambda i: (0, 0)),
        ],
        out_specs=[
            pl.BlockSpec((bm_r, e_num), lambda i: (i, 0)),
            pl.BlockSpec((bm_r, 128), lambda i: (i, 0)),
            pl.BlockSpec((8, e_num), lambda i: (0, 0)),
        ],
        out_shape=[
            jax.ShapeDtypeStruct((n_tok, e_num), jnp.float32),
            jax.ShapeDtypeStruct((n_tok, 128), jnp.float32),
            jax.ShapeDtypeStruct((8, e_num), jnp.float32),
        ],
        scratch_shapes=[pltpu.VMEM((8, e_num), jnp.float32)],
    )(hs, gate_w)

    w01 = meta[:, :2]
    e_all = meta[:, 2:4].astype(jnp.int32).reshape(2 * n_tok)

    n_asg = 2 * n_tok
    rank = meta[:, 4:6].astype(jnp.int32).reshape(n_asg)
    counts = counts_f[0].astype(jnp.int32)

    blocks_per = (counts + _BM - 1) // _BM
    total_blocks = jnp.sum(blocks_per)
    pstart = (jnp.cumsum(blocks_per) - blocks_per) * _BM

    g_map = jnp.repeat(jnp.arange(e_num, dtype=jnp.int32), blocks_per,
                       total_repeat_length=_NB)
    bidx = jnp.arange(_NB, dtype=jnp.int32)
    bs_map = jnp.minimum(bidx, total_blocks - 1)
    v_map = (bidx < total_blocks).astype(jnp.int32)

    pp = pstart[e_all] + rank
    tw = n_tok // _NW
    tc = tw // 2
    pp32 = (pp.reshape(_NW, 2, tc, 2).transpose(0, 1, 3, 2)
            .reshape(4 * _NW, tc))

    mesh = plsc.VectorSubcoreMesh(core_axis_name="c", subcore_axis_name="s")
    x_pad = pl.kernel(
        _dispatch_body,
        out_type=jax.ShapeDtypeStruct((_ROWS_PAD, d), jnp.float32),
        mesh=mesh,
        scratch_types=[
            pltpu.VMEM((4, tc), jnp.int32),
            pltpu.VMEM((tw, d), jnp.float32),
            pltpu.SemaphoreType.DMA,
        ],
    )(hs, pp32)

    out_pad = pl.pallas_call(
        _ffn_body,
        grid_spec=pltpu.PrefetchScalarGridSpec(
            num_scalar_prefetch=3,
            grid=(_NB,),
            in_specs=[
                pl.BlockSpec((_BM, d), lambda i, g, bs, v: (bs[i], 0)),
                pl.BlockSpec((1, f, d), lambda i, g, bs, v: (g[i], 0, 0)),
                pl.BlockSpec((1, f, d), lambda i, g, bs, v: (g[i], 0, 0)),
                pl.BlockSpec((1, d, f), lambda i, g, bs, v: (g[i], 0, 0)),
            ],
            out_specs=pl.BlockSpec((_BM, d), lambda i, g, bs, v: (bs[i], 0)),
        ),
        out_shape=jax.ShapeDtypeStruct((_ROWS_PAD, d), jnp.float32),
    )(g_map, bs_map, v_map, x_pad, w1, w3, w2)

    wb0 = jnp.broadcast_to(w01[:, 0:1], (n_tok, 16))
    wb1 = jnp.broadcast_to(w01[:, 1:2], (n_tok, 16))
    final = pl.kernel(
        _combine_sc_body,
        out_type=jax.ShapeDtypeStruct((n_tok, d), jnp.float32),
        mesh=plsc.VectorSubcoreMesh(core_axis_name="c",
                                    subcore_axis_name="s"),
        scratch_types=[
            pltpu.VMEM((tc,), jnp.int32),
            pltpu.VMEM((tc,), jnp.int32),
            pltpu.VMEM((tc, d), jnp.float32),
            pltpu.VMEM((tc, d), jnp.float32),
            pltpu.VMEM((tc, 16), jnp.float32),
            pltpu.VMEM((tc, 16), jnp.float32),
            pltpu.SemaphoreType.DMA,
        ],
    )(out_pad, pp32, wb0, wb1)

    return final.reshape(batch, seq, d), logits


# speedup vs baseline: 1.0324x; 1.0204x over previous
import functools

import jax
import jax.numpy as jnp
from jax import lax
from jax.experimental import pallas as pl
from jax.experimental.pallas import tpu as pltpu
from jax.experimental.pallas import tpu_sc as plsc

_BETA = 0.2
_BM = 128
_NB = 96
_ROWS_PAD = _NB * _BM

_SC = plsc.get_sparse_core_info()
_NW = _SC.num_cores * _SC.num_subcores


def _router_body(hs_ref, gw_ref, logits_ref, meta_ref, counts_ref, carry_ref):
    i = pl.program_id(0)

    @pl.when(i == 0)
    def _():
        carry_ref[...] = jnp.zeros_like(carry_ref)

    x = hs_ref[...]
    logits = jax.lax.dot_general(
        x, gw_ref[...], (((1,), (1,)), ((), ())),
        preferred_element_type=jnp.float32)
    logits_ref[...] = logits

    mx = jnp.max(logits, axis=1, keepdims=True)
    ex = jnp.exp(logits - mx)
    p = ex / jnp.sum(ex, axis=1, keepdims=True)

    bm, e = p.shape
    idx = jax.lax.broadcasted_iota(jnp.int32, (bm, e), 1)
    p1 = jnp.max(p, axis=1, keepdims=True)
    e0 = jnp.min(jnp.where(p == p1, idx, e), axis=1, keepdims=True)
    pm = jnp.where(idx == e0, -jnp.inf, p)
    p2 = jnp.max(pm, axis=1, keepdims=True)
    e1 = jnp.min(jnp.where(pm == p2, idx, e), axis=1, keepdims=True)

    skip = p2 < _BETA * p1
    denom = p1 + jnp.where(skip, 0.0, p2)
    w0 = p1 / denom
    w1 = jnp.where(skip, 0.0, p2 / denom)

    oh0 = (idx == e0).astype(jnp.float32)
    oh1 = (idx == e1).astype(jnp.float32)
    ohall = oh0 + oh1
    ri = jax.lax.broadcasted_iota(jnp.int32, (bm, bm), 0)
    ci = jax.lax.broadcasted_iota(jnp.int32, (bm, bm), 1)
    tri = (ri > ci).astype(jnp.float32)
    csum_excl = jax.lax.dot_general(tri, ohall, (((1,), (0,)), ((), ())),
                                    preferred_element_type=jnp.float32)
    tot = carry_ref[0:1, :] + csum_excl
    rank0 = jnp.sum(tot * oh0, axis=1, keepdims=True)
    rank1 = jnp.sum(tot * oh1, axis=1, keepdims=True)
    newc = carry_ref[0:1, :] + jnp.sum(ohall, axis=0, keepdims=True)
    carry_ref[0:1, :] = newc
    counts_ref[0:1, :] = newc

    col = jax.lax.broadcasted_iota(jnp.int32, (bm, meta_ref.shape[1]), 1)
    meta = (w0 * (col == 0) + w1 * (col == 1)
            + e0.astype(jnp.float32) * (col == 2)
            + e1.astype(jnp.float32) * (col == 3)
            + rank0 * (col == 4) + rank1 * (col == 5))
    meta_ref[...] = meta


def _ffn_body(g_ref, bs_ref, v_ref, x_ref, w1_ref, w3_ref, w2_ref, out_ref):
    @pl.when(v_ref[pl.program_id(0)] == 1)
    def _():
        x = x_ref[...]
        a = jax.lax.dot_general(x, w1_ref[0], (((1,), (1,)), ((), ())),
                                preferred_element_type=jnp.float32)
        b = jax.lax.dot_general(x, w3_ref[0], (((1,), (1,)), ((), ())),
                                preferred_element_type=jnp.float32)
        h = (a * jax.nn.sigmoid(a)) * b
        out_ref[...] = jax.lax.dot_general(
            h, w2_ref[0], (((1,), (1,)), ((), ())),
            preferred_element_type=jnp.float32)


def _dispatch_body(hs_ref, pp_ref, x_pad_ref, idx0_v, idx1_v, rows_v, sem):
    wid = lax.axis_index("s") * _SC.num_cores + lax.axis_index("c")
    tw = rows_v.shape[0]
    base = wid * tw
    pltpu.sync_copy(pp_ref.at[2 * wid], idx0_v)
    pltpu.sync_copy(pp_ref.at[2 * wid + 1], idx1_v)
    pltpu.sync_copy(hs_ref.at[pl.ds(base, tw)], rows_v)
    c0 = pltpu.async_copy(rows_v, x_pad_ref.at[idx0_v], sem)
    c1 = pltpu.async_copy(rows_v, x_pad_ref.at[idx1_v], sem)
    c0.wait()
    c1.wait()


def _gather_body(out_pad_ref, pp_ref, gcat_ref, idx0_v, idx1_v, buf_v, sem):
    wid = lax.axis_index("s") * _SC.num_cores + lax.axis_index("c")
    tc = buf_v.shape[1]
    tw = 2 * tc
    n_tok = gcat_ref.shape[0] // 2
    base = wid * tw
    pltpu.sync_copy(pp_ref.at[2 * wid], idx0_v)
    pltpu.sync_copy(pp_ref.at[2 * wid + 1], idx1_v)
    idxs = (idx0_v, idx1_v)
    chunks = [(s, h) for s in range(2) for h in range(2)]

    def _gather(k):
        s, h = chunks[k]
        return pltpu.async_copy(
            out_pad_ref.at[idxs[s].at[pl.ds(h * tc, tc)]],
            buf_v.at[k % 2], sem)

    cur = _gather(0)
    for k in range(4):
        s, h = chunks[k]
        cur.wait()
        nxt = _gather(k + 1) if k < 3 else None
        pltpu.sync_copy(buf_v.at[k % 2],
                        gcat_ref.at[pl.ds(s * n_tok + base + h * tc, tc)])
        cur = nxt


def _combine_body(g0_ref, g1_ref, w_ref, f_ref):
    f_ref[...] = (g0_ref[...] * w_ref[:, 0:1] + g1_ref[...] * w_ref[:, 1:2])


def kernel(hidden_states, gate_w, w1, w3, w2):
    batch, seq, d = hidden_states.shape
    n_tok = batch * seq
    e_num = gate_w.shape[0]
    f = w1.shape[1]
    hs = hidden_states.reshape(n_tok, d)

    bm_r = 256
    logits, meta, counts_f = pl.pallas_call(
        _router_body,
        grid=(n_tok // bm_r,),
        in_specs=[
            pl.BlockSpec((bm_r, d), lambda i: (i, 0)),
            pl.BlockSpec((e_num, d), lambda i: (0, 0)),
        ],
        out_specs=[
            pl.BlockSpec((bm_r, e_num), lambda i: (i, 0)),
            pl.BlockSpec((bm_r, 128), lambda i: (i, 0)),
            pl.BlockSpec((8, e_num), lambda i: (0, 0)),
        ],
        out_shape=[
            jax.ShapeDtypeStruct((n_tok, e_num), jnp.float32),
            jax.ShapeDtypeStruct((n_tok, 128), jnp.float32),
            jax.ShapeDtypeStruct((8, e_num), jnp.float32),
        ],
        scratch_shapes=[pltpu.VMEM((8, e_num), jnp.float32)],
    )(hs, gate_w)

    w01 = meta[:, :2]
    e_all = meta[:, 2:4].astype(jnp.int32).reshape(2 * n_tok)

    n_asg = 2 * n_tok
    rank = meta[:, 4:6].astype(jnp.int32).reshape(n_asg)
    counts = counts_f[0].astype(jnp.int32)

    blocks_per = (counts + _BM - 1) // _BM
    total_blocks = jnp.sum(blocks_per)
    pstart = (jnp.cumsum(blocks_per) - blocks_per) * _BM

    g_map = jnp.repeat(jnp.arange(e_num, dtype=jnp.int32), blocks_per,
                       total_repeat_length=_NB)
    bidx = jnp.arange(_NB, dtype=jnp.int32)
    bs_map = jnp.minimum(bidx, total_blocks - 1)
    v_map = (bidx < total_blocks).astype(jnp.int32)

    pp = pstart[e_all] + rank
    tw = n_tok // _NW
    pp_rows = (pp.reshape(_NW, tw, 2).transpose(0, 2, 1)
               .reshape(2 * _NW, tw))

    mesh = plsc.VectorSubcoreMesh(core_axis_name="c", subcore_axis_name="s")
    x_pad = pl.kernel(
        _dispatch_body,
        out_type=jax.ShapeDtypeStruct((_ROWS_PAD, d), jnp.float32),
        mesh=mesh,
        scratch_types=[
            pltpu.VMEM((tw,), jnp.int32),
            pltpu.VMEM((tw,), jnp.int32),
            pltpu.VMEM((tw, d), jnp.float32),
            pltpu.SemaphoreType.DMA,
        ],
    )(hs, pp_rows)

    out_pad = pl.pallas_call(
        _ffn_body,
        grid_spec=pltpu.PrefetchScalarGridSpec(
            num_scalar_prefetch=3,
            grid=(_NB,),
            in_specs=[
                pl.BlockSpec((_BM, d), lambda i, g, bs, v: (bs[i], 0)),
                pl.BlockSpec((1, f, d), lambda i, g, bs, v: (g[i], 0, 0)),
                pl.BlockSpec((1, f, d), lambda i, g, bs, v: (g[i], 0, 0)),
                pl.BlockSpec((1, d, f), lambda i, g, bs, v: (g[i], 0, 0)),
            ],
            out_specs=pl.BlockSpec((_BM, d), lambda i, g, bs, v: (bs[i], 0)),
        ),
        out_shape=jax.ShapeDtypeStruct((_ROWS_PAD, d), jnp.float32),
    )(g_map, bs_map, v_map, x_pad, w1, w3, w2)

    gcat = pl.kernel(
        _gather_body,
        out_type=jax.ShapeDtypeStruct((2 * n_tok, d), jnp.float32),
        mesh=plsc.VectorSubcoreMesh(core_axis_name="c",
                                    subcore_axis_name="s"),
        scratch_types=[
            pltpu.VMEM((tw,), jnp.int32),
            pltpu.VMEM((tw,), jnp.int32),
            pltpu.VMEM((2, tw // 2, d), jnp.float32),
            pltpu.SemaphoreType.DMA,
        ],
    )(out_pad, pp_rows)

    final = pl.pallas_call(
        _combine_body,
        grid=(n_tok // bm_r,),
        in_specs=[
            pl.BlockSpec((bm_r, d), lambda i: (i, 0)),
            pl.BlockSpec((bm_r, d), lambda i: (i + n_tok // bm_r, 0)),
            pl.BlockSpec((bm_r, 2), lambda i: (i, 0)),
        ],
        out_specs=pl.BlockSpec((bm_r, d), lambda i: (i, 0)),
        out_shape=jax.ShapeDtypeStruct((n_tok, d), jnp.float32),
    )(gcat, gcat, w01)

    return final.reshape(batch, seq, d), logits


# speedup vs baseline: 1.0382x; 1.0056x over previous
import jax
import jax.numpy as jnp
from jax import lax
from jax.experimental import pallas as pl
from jax.experimental.pallas import tpu as pltpu
from jax.experimental.pallas import tpu_sc as plsc

_BETA = 0.2
_BM = 128
_NB = 96
_ROWS_PAD = _NB * _BM

_SC = plsc.get_sparse_core_info()
_NW = _SC.num_cores * _SC.num_subcores


def _router_body(hs_ref, gw_ref, logits_ref, meta_ref, counts_ref, carry_ref):
    i = pl.program_id(0)

    @pl.when(i == 0)
    def _():
        carry_ref[...] = jnp.zeros_like(carry_ref)

    x = hs_ref[...]
    logits = jax.lax.dot_general(
        x, gw_ref[...], (((1,), (1,)), ((), ())),
        preferred_element_type=jnp.float32)
    logits_ref[...] = logits

    mx = jnp.max(logits, axis=1, keepdims=True)
    ex = jnp.exp(logits - mx)
    p = ex / jnp.sum(ex, axis=1, keepdims=True)

    bm, e = p.shape
    idx = jax.lax.broadcasted_iota(jnp.int32, (bm, e), 1)
    p1 = jnp.max(p, axis=1, keepdims=True)
    e0 = jnp.min(jnp.where(p == p1, idx, e), axis=1, keepdims=True)
    pm = jnp.where(idx == e0, -jnp.inf, p)
    p2 = jnp.max(pm, axis=1, keepdims=True)
    e1 = jnp.min(jnp.where(pm == p2, idx, e), axis=1, keepdims=True)

    skip = p2 < _BETA * p1
    denom = p1 + jnp.where(skip, 0.0, p2)
    w0 = p1 / denom
    w1 = jnp.where(skip, 0.0, p2 / denom)

    oh0 = (idx == e0).astype(jnp.float32)
    oh1 = (idx == e1).astype(jnp.float32)
    ohall = oh0 + oh1
    ri = jax.lax.broadcasted_iota(jnp.int32, (bm, bm), 0)
    ci = jax.lax.broadcasted_iota(jnp.int32, (bm, bm), 1)
    tri = (ri > ci).astype(jnp.float32)
    csum_excl = jax.lax.dot_general(tri, ohall, (((1,), (0,)), ((), ())),
                                    preferred_element_type=jnp.float32)
    tot = carry_ref[0:1, :] + csum_excl
    rank0 = jnp.sum(tot * oh0, axis=1, keepdims=True)
    rank1 = jnp.sum(tot * oh1, axis=1, keepdims=True)
    newc = carry_ref[0:1, :] + jnp.sum(ohall, axis=0, keepdims=True)
    carry_ref[0:1, :] = newc
    counts_ref[0:1, :] = newc

    col = jax.lax.broadcasted_iota(jnp.int32, (bm, meta_ref.shape[1]), 1)
    meta = (w0 * (col == 0) + w1 * (col == 1)
            + e0.astype(jnp.float32) * (col == 2)
            + e1.astype(jnp.float32) * (col == 3)
            + rank0 * (col == 4) + rank1 * (col == 5))
    meta_ref[...] = meta


def _ffn_body(g_ref, bs_ref, v_ref, x_ref, w1_ref, w3_ref, w2_ref, out_ref):
    @pl.when(v_ref[pl.program_id(0)] == 1)
    def _():
        x = x_ref[...]
        a = jax.lax.dot_general(x, w1_ref[0], (((1,), (1,)), ((), ())),
                                preferred_element_type=jnp.float32)
        b = jax.lax.dot_general(x, w3_ref[0], (((1,), (1,)), ((), ())),
                                preferred_element_type=jnp.float32)
        h = (a * jax.nn.sigmoid(a)) * b
        out_ref[...] = jax.lax.dot_general(
            h, w2_ref[0], (((1,), (1,)), ((), ())),
            preferred_element_type=jnp.float32)


def _dispatch_body(hs_ref, pp_ref, x_pad_ref, idx0_v, idx1_v, rows_v, sem):
    wid = lax.axis_index("s") * _SC.num_cores + lax.axis_index("c")
    tw = rows_v.shape[0]
    base = wid * tw
    pltpu.sync_copy(pp_ref.at[2 * wid], idx0_v)
    pltpu.sync_copy(pp_ref.at[2 * wid + 1], idx1_v)
    pltpu.sync_copy(hs_ref.at[pl.ds(base, tw)], rows_v)
    c0 = pltpu.async_copy(rows_v, x_pad_ref.at[idx0_v], sem)
    c1 = pltpu.async_copy(rows_v, x_pad_ref.at[idx1_v], sem)
    c0.wait()
    c1.wait()


def _gather_body(out_pad_ref, pp_ref, gcat_ref, idx0_v, idx1_v, buf_v, sem):
    wid = lax.axis_index("s") * _SC.num_cores + lax.axis_index("c")
    tc = buf_v.shape[1]
    tw = 2 * tc
    n_tok = gcat_ref.shape[0] // 2
    base = wid * tw
    pltpu.sync_copy(pp_ref.at[2 * wid], idx0_v)
    pltpu.sync_copy(pp_ref.at[2 * wid + 1], idx1_v)
    idxs = (idx0_v, idx1_v)
    chunks = [(s, h) for s in range(2) for h in range(2)]

    def _gather(k):
        s, h = chunks[k]
        return pltpu.async_copy(
            out_pad_ref.at[idxs[s].at[pl.ds(h * tc, tc)]],
            buf_v.at[k % 2], sem)

    cur = _gather(0)
    for k in range(4):
        s, h = chunks[k]
        cur.wait()
        nxt = _gather(k + 1) if k < 3 else None
        pltpu.sync_copy(buf_v.at[k % 2],
                        gcat_ref.at[pl.ds(s * n_tok + base + h * tc, tc)])
        cur = nxt


def _combine_body(g0_ref, g1_ref, w_ref, f_ref):
    f_ref[...] = (g0_ref[...] * w_ref[:, 0:1] + g1_ref[...] * w_ref[:, 1:2])


def kernel(hidden_states, gate_w, w1, w3, w2):
    batch, seq, d = hidden_states.shape
    n_tok = batch * seq
    e_num = gate_w.shape[0]
    f = w1.shape[1]
    hs = hidden_states.reshape(n_tok, d)

    bm_r = 256
    logits, meta, counts_f = pl.pallas_call(
        _router_body,
        grid=(n_tok // bm_r,),
        in_specs=[
            pl.BlockSpec((bm_r, d), lambda i: (i, 0)),
            pl.BlockSpec((e_num, d), lambda i: (0, 0)),
        ],
        out_specs=[
            pl.BlockSpec((bm_r, e_num), lambda i: (i, 0)),
            pl.BlockSpec((bm_r, 128), lambda i: (i, 0)),
            pl.BlockSpec((8, e_num), lambda i: (0, 0)),
        ],
        out_shape=[
            jax.ShapeDtypeStruct((n_tok, e_num), jnp.float32),
            jax.ShapeDtypeStruct((n_tok, 128), jnp.float32),
            jax.ShapeDtypeStruct((8, e_num), jnp.float32),
        ],
        scratch_shapes=[pltpu.VMEM((8, e_num), jnp.float32)],
    )(hs, gate_w)

    w01 = meta[:, :2]
    e_all = meta[:, 2:4].astype(jnp.int32).reshape(2 * n_tok)

    n_asg = 2 * n_tok
    rank = meta[:, 4:6].astype(jnp.int32).reshape(n_asg)
    counts = counts_f[0].astype(jnp.int32)

    blocks_per = (counts + _BM - 1) // _BM
    total_blocks = jnp.sum(blocks_per)
    pstart = (jnp.cumsum(blocks_per) - blocks_per) * _BM

    g_map = jnp.repeat(jnp.arange(e_num, dtype=jnp.int32), blocks_per,
                       total_repeat_length=_NB)
    bidx = jnp.arange(_NB, dtype=jnp.int32)
    bs_map = jnp.minimum(bidx, total_blocks - 1)
    v_map = (bidx < total_blocks).astype(jnp.int32)

    pp = pstart[e_all] + rank
    tw = n_tok // _NW
    pp_rows = (pp.reshape(_NW, tw, 2).transpose(0, 2, 1)
               .reshape(2 * _NW, tw))

    mesh = plsc.VectorSubcoreMesh(core_axis_name="c", subcore_axis_name="s")
    x_pad = pl.kernel(
        _dispatch_body,
        out_type=jax.ShapeDtypeStruct((_ROWS_PAD, d), jnp.float32),
        mesh=mesh,
        scratch_types=[
            pltpu.VMEM((tw,), jnp.int32),
            pltpu.VMEM((tw,), jnp.int32),
            pltpu.VMEM((tw, d), jnp.float32),
            pltpu.SemaphoreType.DMA,
        ],
    )(hs, pp_rows)

    out_pad = pl.pallas_call(
        _ffn_body,
        grid_spec=pltpu.PrefetchScalarGridSpec(
            num_scalar_prefetch=3,
            grid=(_NB,),
            in_specs=[
                pl.BlockSpec((_BM, d), lambda i, g, bs, v: (bs[i], 0)),
                pl.BlockSpec((1, f, d), lambda i, g, bs, v: (g[i], 0, 0)),
                pl.BlockSpec((1, f, d), lambda i, g, bs, v: (g[i], 0, 0)),
                pl.BlockSpec((1, d, f), lambda i, g, bs, v: (g[i], 0, 0)),
            ],
            out_specs=pl.BlockSpec((_BM, d), lambda i, g, bs, v: (bs[i], 0)),
        ),
        out_shape=jax.ShapeDtypeStruct((_ROWS_PAD, d), jnp.float32),
    )(g_map, bs_map, v_map, x_pad, w1, w3, w2)

    gcat = pl.kernel(
        _gather_body,
        out_type=jax.ShapeDtypeStruct((2 * n_tok, d), jnp.float32),
        mesh=plsc.VectorSubcoreMesh(core_axis_name="c",
                                    subcore_axis_name="s"),
        scratch_types=[
            pltpu.VMEM((tw,), jnp.int32),
            pltpu.VMEM((tw,), jnp.int32),
            pltpu.VMEM((2, tw // 2, d), jnp.float32),
            pltpu.SemaphoreType.DMA,
        ],
    )(out_pad, pp_rows)

    final = pl.pallas_call(
        _combine_body,
        grid=(n_tok // bm_r,),
        in_specs=[
            pl.BlockSpec((bm_r, d), lambda i: (i, 0)),
            pl.BlockSpec((bm_r, d), lambda i: (i + n_tok // bm_r, 0)),
            pl.BlockSpec((bm_r, 2), lambda i: (i, 0)),
        ],
        out_specs=pl.BlockSpec((bm_r, d), lambda i: (i, 0)),
        out_shape=jax.ShapeDtypeStruct((n_tok, d), jnp.float32),
    )(gcat, gcat, w01)

    return final.reshape(batch, seq, d), logits
